# initial kernel scaffold (unmeasured)
import jax
import jax.numpy as jnp
from jax import lax
from jax.experimental import pallas as pl
from jax.experimental.pallas import tpu as pltpu


def kernel(
    x,
):
    def body(*refs):
        pass

    out_shape = jax.ShapeDtypeStruct(..., jnp.float32)
    return pl.pallas_call(body, out_shape=out_shape)(...)


# baseline (device time: 54317 ns/iter reference)
import jax
import jax.numpy as jnp
from jax import lax
from jax.experimental import pallas as pl
from jax.experimental.pallas import tpu as pltpu

N_X = 2


def kernel(x):
    m_per, n = x.shape

    def body(x_ref, out_ref, send_sem, recv_sem):
        my_x = lax.axis_index("x")
        my_y = lax.axis_index("y")
        my_z = lax.axis_index("z")
        partner = (1 - my_x, my_y, my_z)

        barrier_sem = pltpu.get_barrier_semaphore()
        pl.semaphore_signal(
            barrier_sem, inc=1, device_id=partner,
            device_id_type=pl.DeviceIdType.MESH,
        )
        pl.semaphore_wait(barrier_sem, 1)

        rdma = pltpu.make_async_remote_copy(
            src_ref=x_ref,
            dst_ref=out_ref.at[pl.ds(my_x * m_per, m_per), :],
            send_sem=send_sem,
            recv_sem=recv_sem,
            device_id=partner,
            device_id_type=pl.DeviceIdType.MESH,
        )
        rdma.start()

        out_ref[pl.ds(my_x * m_per, m_per), :] = x_ref[...]

        rdma.wait()

    return pl.pallas_call(
        body,
        out_shape=jax.ShapeDtypeStruct((N_X * m_per, n), x.dtype),
        in_specs=[pl.BlockSpec(memory_space=pltpu.VMEM)],
        out_specs=pl.BlockSpec(memory_space=pltpu.VMEM),
        scratch_shapes=[
            pltpu.SemaphoreType.DMA,
            pltpu.SemaphoreType.DMA,
        ],
        compiler_params=pltpu.CompilerParams(collective_id=0),
    )(x)


# device time: 37284 ns/iter; 1.4568x vs baseline; 1.4568x over previous
import jax
import jax.numpy as jnp
from jax import lax
from jax.experimental import pallas as pl
from jax.experimental.pallas import tpu as pltpu

N_X = 2
N_CHUNKS = 8


def kernel(x):
    m_per, n = x.shape
    half = m_per // 2
    rpc = half // N_CHUNKS

    def body(x_ref, out_ref, x_send, x_recv, z_send, z_recv):
        my_x = lax.axis_index("x")
        my_z = lax.axis_index("z")
        my_y = lax.axis_index("y")
        px = (1 - my_x, my_y, my_z)
        pz = (my_x, my_y, 1 - my_z)

        barrier_sem = pltpu.get_barrier_semaphore()
        for nbr in (px, pz):
            pl.semaphore_signal(
                barrier_sem, inc=1, device_id=nbr,
                device_id_type=pl.DeviceIdType.MESH,
            )
        pl.semaphore_wait(barrier_sem, 2)

        send_base = my_x * m_per + my_z * half
        recv_x_base = (1 - my_x) * m_per + my_z * half
        recv_z_base = (1 - my_x) * m_per + (1 - my_z) * half

        x_rdmas = []
        for i in range(N_CHUNKS):
            r = i * rpc
            rdma = pltpu.make_async_remote_copy(
                src_ref=x_ref.at[pl.ds(my_z * half + r, rpc), :],
                dst_ref=out_ref.at[pl.ds(send_base + r, rpc), :],
                send_sem=x_send.at[i],
                recv_sem=x_recv.at[i],
                device_id=px,
                device_id_type=pl.DeviceIdType.MESH,
            )
            rdma.start()
            x_rdmas.append(rdma)

        out_ref[pl.ds(my_x * m_per, m_per), :] = x_ref[...]

        z_rdmas = []
        for i in range(N_CHUNKS):
            r = i * rpc
            x_rdmas[i].wait_recv()
            rdma = pltpu.make_async_remote_copy(
                src_ref=out_ref.at[pl.ds(recv_x_base + r, rpc), :],
                dst_ref=out_ref.at[pl.ds(recv_x_base + r, rpc), :],
                send_sem=z_send.at[i],
                recv_sem=z_recv.at[i],
                device_id=pz,
                device_id_type=pl.DeviceIdType.MESH,
            )
            rdma.start()
            z_rdmas.append(rdma)
            _ = recv_z_base

        for i in range(N_CHUNKS):
            z_rdmas[i].wait_recv()
        for i in range(N_CHUNKS):
            x_rdmas[i].wait_send()
            z_rdmas[i].wait_send()

    return pl.pallas_call(
        body,
        out_shape=jax.ShapeDtypeStruct((N_X * m_per, n), x.dtype),
        in_specs=[pl.BlockSpec(memory_space=pltpu.VMEM)],
        out_specs=pl.BlockSpec(memory_space=pltpu.VMEM),
        scratch_shapes=[
            pltpu.SemaphoreType.DMA((N_CHUNKS,)),
            pltpu.SemaphoreType.DMA((N_CHUNKS,)),
            pltpu.SemaphoreType.DMA((N_CHUNKS,)),
            pltpu.SemaphoreType.DMA((N_CHUNKS,)),
        ],
        compiler_params=pltpu.CompilerParams(collective_id=0),
    )(x)


# device time: 29338 ns/iter; 1.8514x vs baseline; 1.2708x over previous
import jax
import jax.numpy as jnp
from jax import lax
from jax.experimental import pallas as pl
from jax.experimental.pallas import tpu as pltpu

N_X = 2
NC = 16
XD = 9
YD = 4
ZD = NC - XD - YD


def kernel(x):
    m_per, n = x.shape
    qtr = m_per // 4
    rpc = qtr // NC

    def body(x_ref, out_ref, x_s, x_r, yf_s, yf_r, zf_s, zf_r,
             ya_s, ya_r, zb_s, zb_r, loc_sem):
        my_x = lax.axis_index("x")
        my_y = lax.axis_index("y")
        my_z = lax.axis_index("z")
        px = (1 - my_x, my_y, my_z)
        py = (my_x, 1 - my_y, my_z)
        pz = (my_x, my_y, 1 - my_z)

        qd = 2 * my_z + my_y
        qdy = 2 * my_z + (1 - my_y)
        qdz = 2 * (1 - my_z) + my_y
        qdiag = 2 * (1 - my_z) + (1 - my_y)

        sb = my_x * m_per
        ob = (1 - my_x) * m_per

        barrier_sem = pltpu.get_barrier_semaphore()
        for nbr in (px, py, pz):
            pl.semaphore_signal(
                barrier_sem, inc=1, device_id=nbr,
                device_id_type=pl.DeviceIdType.MESH,
            )
        pl.semaphore_wait(barrier_sem, 3)

        xs = []
        for c in range(NC + XD):
            r = (qd * qtr + c * rpc if c < NC
                 else qdiag * qtr + (c - NC) * rpc)
            rd = pltpu.make_async_remote_copy(
                src_ref=x_ref.at[pl.ds(r, rpc), :],
                dst_ref=out_ref.at[pl.ds(sb + r, rpc), :],
                send_sem=x_s.at[c], recv_sem=x_r.at[c],
                device_id=px, device_id_type=pl.DeviceIdType.MESH,
            )
            rd.start()
            xs.append(rd)

        loc = pltpu.make_async_copy(
            x_ref, out_ref.at[pl.ds(sb, m_per), :], loc_sem
        )
        loc.start()

        yf, zf = [], []
        for c in range(NC):
            xs[c].wait_recv()
            r = ob + qd * qtr + c * rpc
            a = pltpu.make_async_remote_copy(
                src_ref=out_ref.at[pl.ds(r, rpc), :],
                dst_ref=out_ref.at[pl.ds(r, rpc), :],
                send_sem=yf_s.at[c], recv_sem=yf_r.at[c],
                device_id=py, device_id_type=pl.DeviceIdType.MESH,
            )
            a.start()
            yf.append(a)
            b = pltpu.make_async_remote_copy(
                src_ref=out_ref.at[pl.ds(r, rpc), :],
                dst_ref=out_ref.at[pl.ds(r, rpc), :],
                send_sem=zf_s.at[c], recv_sem=zf_r.at[c],
                device_id=pz, device_id_type=pl.DeviceIdType.MESH,
            )
            b.start()
            zf.append(b)

        ya, zb = [], []
        for c in range(NC):
            zf[c].wait_recv()
            if XD <= c < XD + YD:
                r = ob + qdz * qtr + c * rpc
                d = pltpu.make_async_remote_copy(
                    src_ref=out_ref.at[pl.ds(r, rpc), :],
                    dst_ref=out_ref.at[pl.ds(r, rpc), :],
                    send_sem=ya_s.at[c - XD], recv_sem=ya_r.at[c - XD],
                    device_id=py, device_id_type=pl.DeviceIdType.MESH,
                )
                d.start()
                ya.append(d)
            yf[c].wait_recv()
            if c >= XD + YD:
                r = ob + qdy * qtr + c * rpc
                d = pltpu.make_async_remote_copy(
                    src_ref=out_ref.at[pl.ds(r, rpc), :],
                    dst_ref=out_ref.at[pl.ds(r, rpc), :],
                    send_sem=zb_s.at[c - XD - YD],
                    recv_sem=zb_r.at[c - XD - YD],
                    device_id=pz, device_id_type=pl.DeviceIdType.MESH,
                )
                d.start()
                zb.append(d)

        for c in range(NC, NC + XD):
            xs[c].wait_recv()
        for d in ya:
            d.wait_recv()
        for d in zb:
            d.wait_recv()
        loc.wait()
        for rd in xs:
            rd.wait_send()
        for a in yf:
            a.wait_send()
        for b in zf:
            b.wait_send()
        for d in ya:
            d.wait_send()
        for d in zb:
            d.wait_send()

    return pl.pallas_call(
        body,
        out_shape=jax.ShapeDtypeStruct((N_X * m_per, n), x.dtype),
        in_specs=[pl.BlockSpec(memory_space=pltpu.VMEM)],
        out_specs=pl.BlockSpec(memory_space=pltpu.VMEM),
        scratch_shapes=[
            pltpu.SemaphoreType.DMA((NC + XD,)),
            pltpu.SemaphoreType.DMA((NC + XD,)),
            pltpu.SemaphoreType.DMA((NC,)),
            pltpu.SemaphoreType.DMA((NC,)),
            pltpu.SemaphoreType.DMA((NC,)),
            pltpu.SemaphoreType.DMA((NC,)),
            pltpu.SemaphoreType.DMA((YD,)),
            pltpu.SemaphoreType.DMA((YD,)),
            pltpu.SemaphoreType.DMA((ZD,)),
            pltpu.SemaphoreType.DMA((ZD,)),
            pltpu.SemaphoreType.DMA,
        ],
        compiler_params=pltpu.CompilerParams(collective_id=0),
    )(x)
